# baseline (speedup 1.0000x reference)
import functools

import jax
import jax.numpy as jnp
from jax import lax
from jax.experimental import pallas as pl
from jax.experimental.pallas import tpu as pltpu
from jax.experimental.pallas import tpu_sc as plsc

B, T, N, C, K = 8, 100, 196, 256, 16
ROWS = B * K
NCORES, NSUB = 2, 16
NW = NCORES * NSUB
RPW = ROWS // NW

_mesh = plsc.VectorSubcoreMesh(
    core_axis_name="c", subcore_axis_name="s",
    num_cores=NCORES, num_subcores=NSUB)


@functools.partial(
    pl.kernel,
    out_type=(
        jax.ShapeDtypeStruct((ROWS, N, C), jnp.float32),
        jax.ShapeDtypeStruct((ROWS, C), jnp.float32),
    ),
    mesh=_mesh,
    scratch_types=[
        pltpu.VMEM((NW * 8 + 8,), jnp.int32),
        pltpu.VMEM((RPW, C), jnp.float32),
        pltpu.SemaphoreType.DMA,
        pltpu.SemaphoreType.DMA,
    ],
)
def _sc_gather(flat1d_hbm, flatpad_hbm, patch_hbm, audio_hbm, outp_hbm,
               outa_hbm, aidx, abuf, sp, sa):
    wid = lax.axis_index("s") * NCORES + lax.axis_index("c")
    base = wid * RPW

    pltpu.sync_copy(flatpad_hbm, aidx)

    ah = pltpu.async_copy(
        audio_hbm.at[aidx.at[pl.ds(wid * 8, RPW)]], abuf, sa)

    v16 = aidx[pl.ds(wid * 8, 16)]
    hs = []
    for r in range(RPW):
        t = v16[r]
        hs.append(pltpu.async_copy(
            patch_hbm.at[pl.ds(t, 1)], outp_hbm.at[pl.ds(base + r, 1)], sp))
    for h in hs:
        h.wait()

    ah.wait()
    pltpu.sync_copy(abuf, outa_hbm.at[pl.ds(base, RPW)])


def kernel(top_k_index_sort, patch_feat, audio_feat):
    idx = top_k_index_sort[:, 0, :].astype(jnp.int32)
    flat = (jnp.arange(B, dtype=jnp.int32)[:, None] * T + idx)
    flat_pad = jnp.concatenate(
        [flat.reshape(NW, RPW),
         jnp.zeros((NW, 8 - RPW), jnp.int32)], axis=1).reshape(NW * 8)
    flat_pad = jnp.concatenate([flat_pad, jnp.zeros((8,), jnp.int32)])
    outp, outa = _sc_gather(
        flat.reshape(ROWS),
        flat_pad,
        patch_feat.reshape(B * T, N, C),
        audio_feat.reshape(B * T, C),
    )
    return outp.reshape(B, K, N, C), outa.reshape(B, K, C)


# speedup vs baseline: 4.8204x; 4.8204x over previous
import functools

import jax
import jax.numpy as jnp
from jax import lax
from jax.experimental import pallas as pl
from jax.experimental.pallas import tpu as pltpu
from jax.experimental.pallas import tpu_sc as plsc

B, T, N, C, K = 8, 100, 196, 256, 16
ROWS = B * K
NCORES, NSUB = 2, 16
NW = NCORES * NSUB
RPW = ROWS // NW

_mesh = plsc.VectorSubcoreMesh(
    core_axis_name="c", subcore_axis_name="s",
    num_cores=NCORES, num_subcores=NSUB)


@functools.partial(
    pl.kernel,
    out_type=(
        jax.ShapeDtypeStruct((ROWS, N, C), jnp.float32),
        jax.ShapeDtypeStruct((ROWS, C), jnp.float32),
    ),
    mesh=_mesh,
    scratch_types=[
        pltpu.VMEM((NW * 8 + 8,), jnp.int32),
        pltpu.VMEM((1, N, C), jnp.float32),
        pltpu.VMEM((1, N, C), jnp.float32),
        pltpu.VMEM((RPW, C), jnp.float32),
        pltpu.SemaphoreType.DMA,
        pltpu.SemaphoreType.DMA,
        pltpu.SemaphoreType.DMA,
        pltpu.SemaphoreType.DMA,
        pltpu.SemaphoreType.DMA,
    ],
)
def _sc_gather(flat1d_hbm, flatpad_hbm, patch_hbm, audio_hbm, outp_hbm,
               outa_hbm, aidx, buf0, buf1, abuf, sg0, sg1, sw0, sw1, sa):
    wid = lax.axis_index("s") * NCORES + lax.axis_index("c")
    base = wid * RPW

    pltpu.sync_copy(flatpad_hbm, aidx)

    ah = pltpu.async_copy(
        audio_hbm.at[aidx.at[pl.ds(wid * 8, RPW)]], abuf, sa)

    v16 = aidx[pl.ds(wid * 8, 16)]
    bufs = (buf0, buf1)
    sgs = (sg0, sg1)
    sws = (sw0, sw1)
    gh = [None] * RPW
    wh = [None] * RPW
    for r in range(2):
        gh[r] = pltpu.async_copy(
            patch_hbm.at[pl.ds(v16[r], 1)], bufs[r], sgs[r])
    for r in range(RPW):
        bsel = r % 2
        gh[r].wait()
        wh[r] = pltpu.async_copy(
            bufs[bsel], outp_hbm.at[pl.ds(base + r, 1)], sws[bsel])
        if r + 2 < RPW:
            wh[r].wait()
            gh[r + 2] = pltpu.async_copy(
                patch_hbm.at[pl.ds(v16[r + 2], 1)], bufs[bsel], sgs[bsel])
    wh[RPW - 2].wait()
    wh[RPW - 1].wait()

    ah.wait()
    pltpu.sync_copy(abuf, outa_hbm.at[pl.ds(base, RPW)])


def kernel(top_k_index_sort, patch_feat, audio_feat):
    idx = top_k_index_sort[:, 0, :].astype(jnp.int32)
    flat = (jnp.arange(B, dtype=jnp.int32)[:, None] * T + idx)
    flat_pad = jnp.concatenate(
        [flat.reshape(NW, RPW),
         jnp.zeros((NW, 8 - RPW), jnp.int32)], axis=1).reshape(NW * 8)
    flat_pad = jnp.concatenate([flat_pad, jnp.zeros((8,), jnp.int32)])
    outp, outa = _sc_gather(
        flat.reshape(ROWS),
        flat_pad,
        patch_feat.reshape(B * T, N, C),
        audio_feat.reshape(B * T, C),
    )
    return outp.reshape(B, K, N, C), outa.reshape(B, K, C)


# speedup vs baseline: 23.9835x; 4.9754x over previous
import functools

import jax
import jax.numpy as jnp
from jax import lax
from jax.experimental import pallas as pl
from jax.experimental.pallas import tpu as pltpu
from jax.experimental.pallas import tpu_sc as plsc

B, T, N, C, K = 8, 100, 196, 256, 16
ROWS = B * K
NCORES, NSUB = 2, 16
NW = NCORES * NSUB
RPW = ROWS // NW
WPB = K // RPW

_mesh = plsc.VectorSubcoreMesh(
    core_axis_name="c", subcore_axis_name="s",
    num_cores=NCORES, num_subcores=NSUB)


@functools.partial(
    pl.kernel,
    out_type=(
        jax.ShapeDtypeStruct((B, N, K, C), jnp.float32),
        jax.ShapeDtypeStruct((ROWS, C), jnp.float32),
    ),
    mesh=_mesh,
    scratch_types=[
        pltpu.VMEM((NW * 8 + 8,), jnp.int32),
        pltpu.VMEM((N, C), jnp.float32),
        pltpu.VMEM((N, C), jnp.float32),
        pltpu.VMEM((RPW, C), jnp.float32),
        pltpu.SemaphoreType.DMA,
        pltpu.SemaphoreType.DMA,
        pltpu.SemaphoreType.DMA,
        pltpu.SemaphoreType.DMA,
        pltpu.SemaphoreType.DMA,
    ],
)
def _sc_gather(tpad_hbm, patch_hbm, audio_hbm, outp_hbm, outa_hbm,
               tidx, buf0, buf1, abuf, sg0, sg1, sw0, sw1, sa):
    wid = lax.axis_index("s") * NCORES + lax.axis_index("c")
    base = wid * RPW
    b = wid // WPB
    k0 = (wid % WPB) * RPW

    pltpu.sync_copy(tpad_hbm, tidx)
    v16 = tidx[pl.ds(wid * 8, 16)]

    ah = []
    for r in range(RPW):
        ah.append(pltpu.async_copy(
            audio_hbm.at[v16[r], b, :], abuf.at[r], sa))

    bufs = (buf0, buf1)
    sgs = (sg0, sg1)
    sws = (sw0, sw1)
    gh = [None] * RPW
    wh = [None] * RPW
    for r in range(2):
        gh[r] = pltpu.async_copy(
            patch_hbm.at[v16[r], :, b, :], bufs[r], sgs[r])
    for r in range(RPW):
        bsel = r % 2
        gh[r].wait()
        wh[r] = pltpu.async_copy(
            bufs[bsel], outp_hbm.at[b, :, k0 + r, :], sws[bsel])
        if r + 2 < RPW:
            wh[r].wait()
            gh[r + 2] = pltpu.async_copy(
                patch_hbm.at[v16[r + 2], :, b, :], bufs[bsel], sgs[bsel])
    wh[RPW - 2].wait()
    wh[RPW - 1].wait()

    for h in ah:
        h.wait()
    pltpu.sync_copy(abuf, outa_hbm.at[pl.ds(base, RPW)])


def kernel(top_k_index_sort, patch_feat, audio_feat):
    tvals = top_k_index_sort[:, 0, :].astype(jnp.int32)
    tpad = jnp.concatenate(
        [tvals.reshape(NW, RPW),
         jnp.zeros((NW, 8 - RPW), jnp.int32)], axis=1).reshape(NW * 8)
    tpad = jnp.concatenate([tpad, jnp.zeros((8,), jnp.int32)])
    outp, outa = _sc_gather(
        tpad,
        jnp.transpose(patch_feat, (1, 2, 0, 3)),
        jnp.transpose(audio_feat, (1, 0, 2)),
    )
    return (jnp.transpose(outp, (0, 2, 1, 3)),
            outa.reshape(B, K, C))


# speedup vs baseline: 24.2550x; 1.0113x over previous
import functools

import jax
import jax.numpy as jnp
from jax import lax
from jax.experimental import pallas as pl
from jax.experimental.pallas import tpu as pltpu
from jax.experimental.pallas import tpu_sc as plsc

B, T, N, C, K = 8, 100, 196, 256, 16
ROWS = B * K
NCORES, NSUB = 2, 16
NW = NCORES * NSUB
RPW = ROWS // NW
WPB = K // RPW

_mesh = plsc.VectorSubcoreMesh(
    core_axis_name="c", subcore_axis_name="s",
    num_cores=NCORES, num_subcores=NSUB)


@functools.partial(
    pl.kernel,
    out_type=(
        jax.ShapeDtypeStruct((B, N, K, C), jnp.float32),
        jax.ShapeDtypeStruct((ROWS, C), jnp.float32),
    ),
    mesh=_mesh,
    scratch_types=[
        pltpu.VMEM((NW * 8 + 8,), jnp.int32),
        pltpu.VMEM((N, C // 2), jnp.float32),
        pltpu.VMEM((N, C // 2), jnp.float32),
        pltpu.VMEM((N, C // 2), jnp.float32),
        pltpu.VMEM((N, C // 2), jnp.float32),
        pltpu.VMEM((RPW, C), jnp.float32),
        pltpu.SemaphoreType.DMA,
        pltpu.SemaphoreType.DMA,
        pltpu.SemaphoreType.DMA,
        pltpu.SemaphoreType.DMA,
        pltpu.SemaphoreType.DMA,
        pltpu.SemaphoreType.DMA,
        pltpu.SemaphoreType.DMA,
        pltpu.SemaphoreType.DMA,
        pltpu.SemaphoreType.DMA,
    ],
)
def _sc_gather(tpad_hbm, patch_hbm, audio_hbm, outp_hbm, outa_hbm,
               tidx, buf0, buf1, buf2, buf3, abuf,
               sg0, sg1, sg2, sg3, sw0, sw1, sw2, sw3, sa):
    wid = lax.axis_index("s") * NCORES + lax.axis_index("c")
    base = wid * RPW
    b = wid // WPB
    k0 = (wid % WPB) * RPW

    pltpu.sync_copy(tpad_hbm, tidx)
    v16 = tidx[pl.ds(wid * 8, 16)]

    ah = []
    for r in range(RPW):
        ah.append(pltpu.async_copy(
            audio_hbm.at[v16[r], b, :], abuf.at[r], sa))

    NU = 2 * RPW
    bufs = (buf0, buf1, buf2, buf3)
    sgs = (sg0, sg1, sg2, sg3)
    sws = (sw0, sw1, sw2, sw3)

    def src_slice(u):
        r, h = u >> 1, u & 1
        return patch_hbm.at[v16[r], :, b, pl.ds(h * (C // 2), C // 2)]

    def dst_slice(u):
        r, h = u >> 1, u & 1
        return outp_hbm.at[b, :, k0 + r, pl.ds(h * (C // 2), C // 2)]

    gh = [None] * NU
    wh = [None] * NU
    for u in range(4):
        gh[u] = pltpu.async_copy(src_slice(u), bufs[u], sgs[u])
    for u in range(NU):
        bsel = u % 4
        gh[u].wait()
        wh[u] = pltpu.async_copy(bufs[bsel], dst_slice(u), sws[bsel])
        if u + 4 < NU:
            wh[u].wait()
            gh[u + 4] = pltpu.async_copy(
                src_slice(u + 4), bufs[bsel], sgs[bsel])
    for u in range(NU - 4, NU):
        wh[u].wait()

    for h in ah:
        h.wait()
    pltpu.sync_copy(abuf, outa_hbm.at[pl.ds(base, RPW)])


def kernel(top_k_index_sort, patch_feat, audio_feat):
    tvals = top_k_index_sort[:, 0, :].astype(jnp.int32)
    tpad = jnp.concatenate(
        [tvals.reshape(NW, RPW),
         jnp.zeros((NW, 8 - RPW), jnp.int32)], axis=1).reshape(NW * 8)
    tpad = jnp.concatenate([tpad, jnp.zeros((8,), jnp.int32)])
    outp, outa = _sc_gather(
        tpad,
        jnp.transpose(patch_feat, (1, 2, 0, 3)),
        jnp.transpose(audio_feat, (1, 0, 2)),
    )
    return (jnp.transpose(outp, (0, 2, 1, 3)),
            outa.reshape(B, K, C))
